# baseline (speedup 1.0000x reference)
import functools

import numpy as np
import jax
import jax.numpy as jnp
from jax import lax
from jax.experimental import pallas as pl
from jax.experimental.pallas import tpu as pltpu
from jax.experimental.pallas import tpu_sc as plsc

_LANES = 16


def _pos_encoding(length, d_model, n=10000):
    d2 = d_model / 2
    position = np.arange(length)[:, np.newaxis]
    index = np.arange(int(d2))[np.newaxis, :]
    angle = position * np.power(n, -index / d2)
    return np.concatenate([np.sin(angle), np.cos(angle)], axis=-1).astype(np.float32)


def kernel(inputs, table):
    B, T = inputs.shape
    V, D = table.shape
    NW = 32
    SW = B // NW
    n_lane = D // _LANES
    scale = float(np.sqrt(D))
    TP = 224
    G1 = 112
    NI = SW * T + 2 * _LANES

    pe = jnp.asarray(_pos_encoding(T, D))
    idx_flat = inputs.reshape(-1).astype(jnp.int32)
    table_p = jnp.pad(table, ((0, 0), (0, D)))

    mesh = plsc.VectorSubcoreMesh(core_axis_name="c", subcore_axis_name="s")

    @functools.partial(
        pl.kernel,
        mesh=mesh,
        out_type=jax.ShapeDtypeStruct((B, T, D), jnp.float32),
        scratch_types=[
            pltpu.VMEM((NI,), jnp.int32),
            pltpu.VMEM((T, D), jnp.float32),
            pltpu.VMEM((TP, 2 * D), jnp.float32),
            pltpu.VMEM((TP, 2 * D), jnp.float32),
            pltpu.VMEM((T, D), jnp.float32),
            pltpu.SemaphoreType.DMA,
            pltpu.SemaphoreType.DMA,
        ],
    )
    def emb(idx_hbm, table_hbm, pe_hbm, out_hbm, idx_v, pe_v, rows0_v, rows1_v,
            out_v, sem0, sem1):
        cid = lax.axis_index("c")
        sid = lax.axis_index("s")
        wid = sid * 2 + cid
        base_seq = wid * SW
        zeros = jnp.zeros((_LANES,), jnp.int32)
        idx_v[pl.ds(NI - 2 * _LANES, _LANES)] = zeros
        idx_v[pl.ds(NI - _LANES, _LANES)] = zeros
        pltpu.sync_copy(idx_hbm.at[pl.ds(base_seq * T, SW * T)],
                        idx_v.at[pl.ds(0, SW * T)])
        pltpu.sync_copy(pe_hbm, pe_v)

        def gathers(s, rows_v, sem):
            for g in range(2):
                pltpu.async_copy(
                    table_hbm.at[idx_v.at[pl.ds(s * T + g * G1, G1)]],
                    rows_v.at[pl.ds(g * G1, G1)], sem)

        def drain(rows_v, sem):
            pltpu.make_async_copy(table_hbm.at[pl.ds(0, TP)], rows_v, sem).wait()

        def process(s, rows_v):
            def row_body(t2, c3):
                for dt in range(2):
                    t = 2 * t2 + dt
                    for l in range(n_lane):
                        sl = pl.ds(l * _LANES, _LANES)
                        out_v[t, sl] = rows_v[t, sl] * scale + pe_v[t, sl]
                return c3

            lax.fori_loop(0, T // 2, row_body, 0)
            pltpu.sync_copy(out_v, out_hbm.at[base_seq + s])

        gathers(0, rows0_v, sem0)

        def pair_body(s2, carry):
            s_even = 2 * s2
            gathers(s_even + 1, rows1_v, sem1)
            drain(rows0_v, sem0)
            process(s_even, rows0_v)

            @pl.when(s2 < SW // 2 - 1)
            def _():
                gathers(s_even + 2, rows0_v, sem0)

            drain(rows1_v, sem1)
            process(s_even + 1, rows1_v)
            return carry

        lax.fori_loop(0, SW // 2, pair_body, 0)

    return emb(idx_flat, table_p, pe)
